# Initial kernel scaffold; baseline (speedup 1.0000x reference)
import jax
import jax.numpy as jnp
from jax.experimental import pallas as pl


def kernel(x):
    raise NotImplementedError("write your pallas kernel here")


# speedup vs baseline: 2.9703x; 2.9703x over previous
import jax
import jax.numpy as jnp
from jax import lax
from jax.experimental import pallas as pl
from jax.experimental.pallas import tpu as pltpu
from jax.experimental.pallas import tpu_sc as plsc

N_ROWS = 4096
N_COLS = 8192
TOPK = 6
NC = 2
NS = 16
NW = NC * NS
COLS_PER_TILE = N_COLS // NW
GROUPS = COLS_PER_TILE // 16
CHUNK_R = 64
N_CHUNKS = N_ROWS // CHUNK_R

_mesh = plsc.VectorSubcoreMesh(core_axis_name="c", subcore_axis_name="s")


def _body(x_hbm, out_hbm, buf0, buf1, acc, outv, sem0, sem1):
    wid = lax.axis_index("s") * NC + lax.axis_index("c")
    c0 = wid * COLS_PER_TILE
    cslice = pl.ds(c0, COLS_PER_TILE)

    inf = jnp.full((16,), jnp.inf, dtype=jnp.float32)

    def init_body(i, _):
        acc[i, :] = inf
        return 0

    lax.fori_loop(0, TOPK * GROUPS, init_body, 0)

    def process(buf):
        for g in range(GROUPS):
            a = tuple(acc[TOPK * g + j, :] for j in range(TOPK))

            def row_body(r, a, g=g):
                v = buf[r, pl.ds(16 * g, 16)]
                out = []
                for j in range(TOPK):
                    lo = jnp.minimum(a[j], v)
                    v = jnp.maximum(a[j], v)
                    out.append(lo)
                return tuple(out)

            a = lax.fori_loop(0, CHUNK_R, row_body, a)
            for j in range(TOPK):
                acc[TOPK * g + j, :] = a[j]

    pltpu.async_copy(x_hbm.at[pl.ds(0, CHUNK_R), cslice], buf0, sem0)

    def pair_body(i, _):
        r0 = i * (2 * CHUNK_R)
        pltpu.async_copy(x_hbm.at[pl.ds(r0 + CHUNK_R, CHUNK_R), cslice],
                         buf1, sem1)
        pltpu.make_async_copy(x_hbm.at[pl.ds(r0, CHUNK_R), cslice],
                              buf0, sem0).wait()
        process(buf0)

        @pl.when(i < N_CHUNKS // 2 - 1)
        def _():
            pltpu.async_copy(
                x_hbm.at[pl.ds(r0 + 2 * CHUNK_R, CHUNK_R), cslice],
                buf0, sem0)

        pltpu.make_async_copy(x_hbm.at[pl.ds(r0 + CHUNK_R, CHUNK_R), cslice],
                              buf1, sem1).wait()
        process(buf1)
        return 0

    lax.fori_loop(0, N_CHUNKS // 2, pair_body, 0)

    s = jnp.zeros((16,), dtype=jnp.float32)
    for g in range(GROUPS):
        for j in range(TOPK):
            s = s + jnp.maximum(acc[TOPK * g + j, :], 1e-8)
    outv[:] = s
    pltpu.sync_copy(outv, out_hbm.at[wid])


_partials = pl.kernel(
    _body,
    out_type=jax.ShapeDtypeStruct((NW, 16), jnp.float32),
    mesh=_mesh,
    scratch_types=[
        pltpu.VMEM((CHUNK_R, COLS_PER_TILE), jnp.float32),
        pltpu.VMEM((CHUNK_R, COLS_PER_TILE), jnp.float32),
        pltpu.VMEM((TOPK * GROUPS, 16), jnp.float32),
        pltpu.VMEM((16,), jnp.float32),
        pltpu.SemaphoreType.DMA,
        pltpu.SemaphoreType.DMA,
    ],
)


def kernel(x):
    parts = _partials(x)
    return jnp.sum(parts) / (N_COLS * TOPK)
